# baseline (speedup 1.0000x reference)
import functools

import jax
import jax.numpy as jnp
from jax import lax
from jax.experimental import pallas as pl
from jax.experimental.pallas import tpu as pltpu
from jax.experimental.pallas import tpu_sc as plsc

N_ROWS = 1_000_000
EMBED_DIM = 64
BATCH = 16384

_info = plsc.get_sparse_core_info()
_NC, _NS = _info.num_cores, _info.num_subcores
_NW = _NC * _NS
_B_PER_W = BATCH // _NW
_CHUNK = 128
_N_CHUNKS = _B_PER_W // _CHUNK


@functools.partial(
    pl.kernel,
    mesh=plsc.VectorSubcoreMesh(core_axis_name="c", subcore_axis_name="s"),
    out_type=jax.ShapeDtypeStruct((BATCH, EMBED_DIM), jnp.float32),
    scratch_types=[
        pltpu.VMEM((_N_CHUNKS, _CHUNK), jnp.int32),
        pltpu.VMEM((_B_PER_W, EMBED_DIM), jnp.float32),
        pltpu.SemaphoreType.DMA,
    ],
    compiler_params=pltpu.CompilerParams(use_tc_tiling_on_sc=False),
)
def _gather_kernel(ids_hbm, table_hbm, out_hbm, idx_v, rows_v, sem):
    wid = lax.axis_index("s") * _NC + lax.axis_index("c")
    pltpu.sync_copy(ids_hbm.at[pl.ds(wid * _N_CHUNKS, _N_CHUNKS)], idx_v)
    copies = []
    for j in range(_N_CHUNKS):
        copies.append(
            pltpu.async_copy(
                table_hbm.at[idx_v.at[j]],
                rows_v.at[pl.ds(j * _CHUNK, _CHUNK)],
                sem,
            )
        )
    for c in copies:
        c.wait()
    pltpu.sync_copy(rows_v, out_hbm.at[pl.ds(wid * _B_PER_W, _B_PER_W)])


def kernel(ids, table):
    return _gather_kernel(ids.reshape(_NW * _N_CHUNKS, _CHUNK), table)


# speedup vs baseline: 1.7289x; 1.7289x over previous
import functools

import jax
import jax.numpy as jnp
from jax import lax
from jax.experimental import pallas as pl
from jax.experimental.pallas import tpu as pltpu
from jax.experimental.pallas import tpu_sc as plsc

N_ROWS = 1_000_000
EMBED_DIM = 64
BATCH = 16384

_info = plsc.get_sparse_core_info()
_NC, _NS = _info.num_cores, _info.num_subcores
_NW = _NC * _NS
_B_PER_W = BATCH // _NW


@functools.partial(
    pl.kernel,
    mesh=plsc.VectorSubcoreMesh(core_axis_name="c", subcore_axis_name="s"),
    out_type=jax.ShapeDtypeStruct((BATCH, EMBED_DIM), jnp.float32),
    scratch_types=[
        pltpu.VMEM((_B_PER_W,), jnp.int32),
        pltpu.VMEM((_B_PER_W, EMBED_DIM), jnp.float32),
        pltpu.SemaphoreType.DMA,
    ],
)
def _gather_kernel(ids_hbm, table_hbm, out_hbm, idx_v, rows_v, sem):
    wid = lax.axis_index("s") * _NC + lax.axis_index("c")
    base = wid * _B_PER_W
    pltpu.sync_copy(ids_hbm.at[pl.ds(base, _B_PER_W)], idx_v)

    def body(g, carry):
        vec = idx_v[pl.ds(g * 16, 16)]
        for k in range(16):
            rid = vec[k]
            pltpu.async_copy(table_hbm.at[rid], rows_v.at[g * 16 + k], sem)
        return carry

    lax.fori_loop(0, _B_PER_W // 16, body, 0)
    pltpu.make_async_copy(
        table_hbm.at[pl.ds(0, _B_PER_W)], rows_v, sem
    ).wait()
    pltpu.sync_copy(rows_v, out_hbm.at[pl.ds(base, _B_PER_W)])


def kernel(ids, table):
    return _gather_kernel(ids.reshape(BATCH), table)


# speedup vs baseline: 3.2027x; 1.8525x over previous
import functools

import jax
import jax.numpy as jnp
from jax import lax
from jax.experimental import pallas as pl
from jax.experimental.pallas import tpu as pltpu
from jax.experimental.pallas import tpu_sc as plsc

N_ROWS = 1_000_000
EMBED_DIM = 64
BATCH = 16384

_info = plsc.get_sparse_core_info()
_NC, _NS = _info.num_cores, _info.num_subcores
_NW = _NC * _NS
_NQ = 4
_NTC = 8
_B_PER_Q = BATCH // _NQ
_BB = 32
_NBATCH = _B_PER_Q // _BB


@functools.partial(
    pl.kernel,
    mesh=plsc.VectorSubcoreMesh(core_axis_name="c", subcore_axis_name="s"),
    out_type=jax.ShapeDtypeStruct((EMBED_DIM, BATCH), jnp.float32),
    scratch_types=[
        pltpu.VMEM((_B_PER_Q,), jnp.int32),
        pltpu.VMEM((2, _BB, 8, 128), jnp.float32),
        pltpu.VMEM((8, _B_PER_Q), jnp.float32),
        pltpu.VMEM((8, 128), jnp.float32),
        pltpu.SemaphoreType.DMA,
    ],
    compiler_params=pltpu.CompilerParams(needs_layout_passes=False),
)
def _gather_kernel(
    ids_hbm, table_t_hbm, out_t_hbm, idx_v, tiles_v, out_vt, drain_v, sem
):
    wid = lax.axis_index("s") * _NC + lax.axis_index("c")
    q = wid >> 3
    tc = wid & 7
    c0 = tc * 8
    idbase = q * _B_PER_Q
    pltpu.sync_copy(ids_hbm.at[pl.ds(idbase, _B_PER_Q)], idx_v)

    lane_iota = lax.iota(jnp.int32, 16)
    sub_idx = lane_iota & 7
    pair_sel = lane_iota < 8

    def fetch_batch(g):
        buf = g & 1
        for k2 in range(_BB // 16):
            v = idx_v[pl.ds(g * _BB + k2 * 16, 16)]
            for k in range(16):
                rb = v[k] >> 7
                pltpu.async_copy(
                    table_t_hbm.at[pl.ds(c0, 8), pl.ds(rb * 128, 128)],
                    tiles_v.at[buf, k2 * 16 + k],
                    sem,
                )

    def extract_batch(h):
        buf = h & 1
        for _ in range(_BB):
            pltpu.make_async_copy(
                table_t_hbm.at[pl.ds(c0, 8), pl.ds(0, 128)], drain_v, sem
            ).wait()
        for k2 in range(_BB // 16):
            lanes = idx_v[pl.ds(h * _BB + k2 * 16, 16)] & 127
            for p in range(8):
                slot = jnp.where(
                    pair_sel, k2 * 16 + 2 * p, k2 * 16 + 2 * p + 1
                )
                lane = jnp.where(pair_sel, lanes[2 * p], lanes[2 * p + 1])
                vals = plsc.load_gather(
                    tiles_v,
                    [jnp.full((16,), buf, jnp.int32), slot, sub_idx, lane],
                )
                bidx = h * _BB + k2 * 16 + 2 * p + jnp.where(
                    pair_sel, 0, 1
                )
                plsc.store_scatter(out_vt, [sub_idx, bidx], vals)

    def body(g, carry):
        @pl.when(g < _NBATCH)
        def _():
            fetch_batch(g)

        @pl.when(g > 0)
        def _():
            extract_batch(g - 1)

        return carry

    lax.fori_loop(0, _NBATCH + 1, body, 0)
    pltpu.sync_copy(
        out_vt, out_t_hbm.at[pl.ds(c0, 8), pl.ds(idbase, _B_PER_Q)]
    )


def kernel(ids, table):
    out_t = _gather_kernel(ids.reshape(BATCH), table.T)
    return out_t.T
